# baseline (device time: 351893 ns/iter reference)
import jax
import jax.numpy as jnp
from jax import lax
from jax.experimental import pallas as pl
from jax.experimental.pallas import tpu as pltpu

N_DEV = 4
TC = 256

_MESH = pl.DeviceIdType.MESH
_ANY = pl.MemorySpace.ANY
_BF = jnp.bfloat16
_F32 = jnp.float32


def kernel(partial, resid, gamma):
    _, M, D = partial.shape
    C = M // N_DEV
    C2 = C // 2
    H = D // 2
    gamma2 = gamma.reshape(1, D)

    def body(partial_ref, resid_ref, gamma_ref, dummy_ref, out_ref,
             acc, commR, commL, stR, stL, cvR, cvL,
             sendR, recvR, sendL, recvL,
             dmaR, dmaL, outR, outL, creditR, creditL):
        p = lax.axis_index("i")
        right = lax.rem(p + 1, N_DEV)
        left = lax.rem(p + N_DEV - 1, N_DEV)

        barrier = pltpu.get_barrier_semaphore()
        for nbr in (left, right):
            pl.semaphore_signal(barrier, inc=1, device_id=(nbr,),
                                device_id_type=_MESH)
        pl.semaphore_wait(barrier, 2)

        half0 = pl.ds(0, H)
        half1 = pl.ds(H, H)

        def ck(k):
            return lax.rem(p + k + 2 * N_DEV, N_DEV)

        def subrows(chunk, s):
            return pl.ds(chunk * C + s * C2, C2)

        class Ring:

            def __init__(self, comm, send_sems, recv_sems, st, cv, dma_sem,
                         credit_sem, target, upstream, col,
                         init_chunk, rs_chunk, ag_chunks):
                self.comm, self.send_sems, self.recv_sems = (
                    comm, send_sems, recv_sems)
                self.st, self.cv = st, cv
                self.dma_sem, self.credit_sem = dma_sem, credit_sem
                self.target, self.upstream, self.col = target, upstream, col
                self.init_chunk, self.rs_chunk, self.ag_chunks = (
                    init_chunk, rs_chunk, ag_chunks)
                self.rd = {}
                self.pf = None

            def src_for(self, j):
                if j in (0, 1, 6, 7):
                    return self.cv.at[j % 2]
                return self.comm.at[(j - 2) % 4]

            def start(self, j):
                r = pltpu.make_async_remote_copy(
                    src_ref=self.src_for(j),
                    dst_ref=self.comm.at[j % 4],
                    send_sem=self.send_sems.at[j % 4],
                    recv_sem=self.recv_sems.at[j % 4],
                    device_id=(self.target,), device_id_type=_MESH)
                self.rd[j] = r
                r.start()

            def wait_recv(self, j):
                self.rd[j].wait_recv()

            def wait_send(self, j):
                self.rd[j].wait_send()

            def prefetch(self, src_rows_ref):
                cp = pltpu.make_async_copy(src_rows_ref, self.st,
                                           self.dma_sem)
                cp.start()
                self.pf = cp

            def pf_accum(self, j):
                self.prefetch(partial_ref.at[
                    0, subrows(self.rs_chunk(j // 2), j % 2), self.col])

            def accum(self, j):
                self.pf.wait()
                self.comm[j % 4] = (
                    self.comm[j % 4].astype(_F32) + self.st[...]
                ).astype(_BF)

            def store(self, j):
                chunk = self.ag_chunks[(j - 6) // 2]
                self.st[...] = self.comm[j % 4].astype(_F32)
                cp = pltpu.make_async_copy(
                    self.st,
                    out_ref.at[subrows(chunk, j % 2), self.col],
                    self.dma_sem)
                cp.start()
                cp.wait()

            def sig(self):
                pl.semaphore_signal(self.credit_sem, inc=1,
                                    device_id=(self.upstream,),
                                    device_id_type=_MESH)

            def take(self):
                pl.semaphore_wait(self.credit_sem, 1)

        R = Ring(commR, sendR, recvR, stR, cvR, dmaR, creditR,
                 target=right, upstream=left, col=half0,
                 init_chunk=ck(0), rs_chunk=lambda h: ck(-h - 1),
                 ag_chunks=[ck(0), ck(-1), ck(2)])
        L = Ring(commL, sendL, recvL, stL, cvL, dmaL, creditL,
                 target=left, upstream=right, col=half1,
                 init_chunk=ck(2), rs_chunk=lambda h: ck(h + 3),
                 ag_chunks=[ck(2), ck(3), ck(0)])
        rings = (R, L)

        for r in rings:
            r.prefetch(partial_ref.at[0, subrows(r.init_chunk, 0), r.col])
        for r in rings:
            r.pf.wait()
            r.cv[0] = r.st[...].astype(_BF)
            r.start(0)
            r.prefetch(partial_ref.at[0, subrows(r.init_chunk, 1), r.col])
        for r in rings:
            r.pf.wait()
            r.cv[1] = r.st[...].astype(_BF)
            r.start(1)
            r.pf_accum(0)

        for r in rings:
            r.wait_recv(0)
            r.accum(0)
            r.start(2)
            r.pf_accum(1)
        for r in rings:
            r.wait_recv(1)
            r.accum(1)
            r.start(3)
            r.pf_accum(2)
        for r in rings:
            r.wait_recv(2)
            r.accum(2)
            r.wait_send(2)
            r.sig()
            r.pf_accum(3)
        own = ck(1)
        for r in rings:
            r.wait_recv(3)
            r.accum(3)
            r.wait_send(3)
            r.sig()
            r.prefetch(partial_ref.at[0, subrows(own, 0), r.col])
        for r in rings:
            r.wait_send(0)
            r.take()
            r.start(4)
        for r in rings:
            r.wait_send(1)
            r.take()
            r.start(5)

        def epilogue(s):
            rsub = pl.ds(s * C2, C2)
            R.pf.wait()
            acc[rsub, half0] = commR[s].astype(_F32) + stR[...]
            R.prefetch(resid_ref.at[subrows(own, s), half0])
            L.pf.wait()
            acc[rsub, half1] = commL[s].astype(_F32) + stL[...]
            L.prefetch(resid_ref.at[subrows(own, s), half1])
            R.pf.wait()
            acc[rsub, half0] = acc[rsub, half0] + stR[...]
            L.pf.wait()
            acc[rsub, half1] = acc[rsub, half1] + stL[...]
            for t in range(C2 // TC):
                tr = pl.ds(s * C2 + t * TC, TC)
                y = acc[tr, :]
                ms = jnp.mean(y * y, axis=1, keepdims=True)
                acc[tr, :] = y * lax.rsqrt(ms + 1e-6) * gamma_ref[...]

        for r in rings:
            r.wait_recv(4)
        epilogue(0)
        for r in rings:
            r.wait_send(4)
            r.sig()
        for r in rings:
            r.cv[0] = acc[pl.ds(0, C2), r.col].astype(_BF)
            r.take()
            r.start(6)
            r.prefetch(partial_ref.at[0, subrows(own, 1), r.col])
        own0 = pltpu.make_async_copy(acc.at[pl.ds(0, C2), :],
                                     out_ref.at[subrows(own, 0), :], outR)
        own0.start()
        for r in rings:
            r.wait_recv(5)
        epilogue(1)
        for r in rings:
            r.wait_send(5)
            r.sig()
            r.sig()
        for r in rings:
            r.cv[1] = acc[pl.ds(C2, C2), r.col].astype(_BF)
            r.take()
            r.start(7)
        own1 = pltpu.make_async_copy(acc.at[pl.ds(C2, C2), :],
                                     out_ref.at[subrows(own, 1), :], outL)
        own1.start()
        for r in rings:
            r.sig()

        for r in rings:
            r.wait_recv(6)
            r.take()
            r.start(8)
            r.store(6)
        for r in rings:
            r.wait_recv(7)
            r.take()
            r.start(9)
            r.store(7)
        for r in rings:
            r.wait_send(8)
            r.sig()
        for r in rings:
            r.wait_recv(8)
            r.wait_send(6)
            r.take()
            r.start(10)
            r.store(8)
        for r in rings:
            r.wait_send(9)
            r.sig()
        for r in rings:
            r.wait_recv(9)
            r.wait_send(7)
            r.take()
            r.start(11)
            r.store(9)
        for r in rings:
            r.wait_recv(10)
            r.store(10)
        for r in rings:
            r.wait_recv(11)
            r.store(11)
        for r in rings:
            r.wait_send(10)
            r.wait_send(11)
        own0.wait()
        own1.wait()

    return pl.pallas_call(
        body,
        out_shape=jax.ShapeDtypeStruct((M, D), jnp.float32),
        in_specs=[
            pl.BlockSpec(memory_space=_ANY),
            pl.BlockSpec(memory_space=_ANY),
            pl.BlockSpec(memory_space=pltpu.VMEM),
            pl.BlockSpec(memory_space=_ANY),
        ],
        out_specs=pl.BlockSpec(memory_space=_ANY),
        input_output_aliases={3: 0},
        scratch_shapes=[
            pltpu.VMEM((C, D), _F32),
            pltpu.VMEM((4, C2, H), _BF),
            pltpu.VMEM((4, C2, H), _BF),
            pltpu.VMEM((C2, H), _F32),
            pltpu.VMEM((C2, H), _F32),
            pltpu.VMEM((2, C2, H), _BF),
            pltpu.VMEM((2, C2, H), _BF),
            pltpu.SemaphoreType.DMA((4,)),
            pltpu.SemaphoreType.DMA((4,)),
            pltpu.SemaphoreType.DMA((4,)),
            pltpu.SemaphoreType.DMA((4,)),
            pltpu.SemaphoreType.DMA,
            pltpu.SemaphoreType.DMA,
            pltpu.SemaphoreType.DMA,
            pltpu.SemaphoreType.DMA,
            pltpu.SemaphoreType.REGULAR,
            pltpu.SemaphoreType.REGULAR,
        ],
        compiler_params=pltpu.CompilerParams(
            collective_id=0,
            vmem_limit_bytes=62 * 1024 * 1024,
        ),
    )(partial, resid, gamma2, jnp.zeros((M, D), _F32))


# device time: 329053 ns/iter; 1.0694x vs baseline; 1.0694x over previous
import jax
import jax.numpy as jnp
from jax import lax
from jax.experimental import pallas as pl
from jax.experimental.pallas import tpu as pltpu

N_DEV = 4
TC = 256

_MESH = pl.DeviceIdType.MESH
_ANY = pl.MemorySpace.ANY
_BF = jnp.bfloat16
_F32 = jnp.float32


def kernel(partial, resid, gamma):
    _, M, D = partial.shape
    C = M // N_DEV
    C2 = C // 2
    H = D // 2
    gamma2 = gamma.reshape(1, D)

    def body(partial_ref, resid_ref, gamma_ref, out_ref,
             acc, commR, commL, stR, stL, cvR, cvL,
             sendR, recvR, sendL, recvL,
             dmaR, dmaL, outR, outL, creditR, creditL):
        p = lax.axis_index("i")
        right = lax.rem(p + 1, N_DEV)
        left = lax.rem(p + N_DEV - 1, N_DEV)

        barrier = pltpu.get_barrier_semaphore()
        for nbr in (left, right):
            pl.semaphore_signal(barrier, inc=1, device_id=(nbr,),
                                device_id_type=_MESH)
        pl.semaphore_wait(barrier, 2)

        half0 = pl.ds(0, H)
        half1 = pl.ds(H, H)

        def ck(k):
            return lax.rem(p + k + 2 * N_DEV, N_DEV)

        def subrows(chunk, s):
            return pl.ds(chunk * C + s * C2, C2)

        class Ring:

            def __init__(self, comm, send_sems, recv_sems, st, cv, dma_sem,
                         credit_sem, target, upstream, col,
                         init_chunk, rs_chunk, ag_chunks):
                self.comm, self.send_sems, self.recv_sems = (
                    comm, send_sems, recv_sems)
                self.st, self.cv = st, cv
                self.dma_sem, self.credit_sem = dma_sem, credit_sem
                self.target, self.upstream, self.col = target, upstream, col
                self.init_chunk, self.rs_chunk, self.ag_chunks = (
                    init_chunk, rs_chunk, ag_chunks)
                self.rd = {}
                self.pf = None

            def src_for(self, j):
                if j in (0, 1, 6, 7):
                    return self.cv.at[j % 2]
                return self.comm.at[(j - 2) % 4]

            def start(self, j):
                r = pltpu.make_async_remote_copy(
                    src_ref=self.src_for(j),
                    dst_ref=self.comm.at[j % 4],
                    send_sem=self.send_sems.at[j % 4],
                    recv_sem=self.recv_sems.at[j % 4],
                    device_id=(self.target,), device_id_type=_MESH)
                self.rd[j] = r
                r.start()

            def wait_recv(self, j):
                self.rd[j].wait_recv()

            def wait_send(self, j):
                self.rd[j].wait_send()

            def prefetch(self, src_rows_ref):
                cp = pltpu.make_async_copy(src_rows_ref, self.st,
                                           self.dma_sem)
                cp.start()
                self.pf = cp

            def pf_accum(self, j):
                self.prefetch(partial_ref.at[
                    0, subrows(self.rs_chunk(j // 2), j % 2), self.col])

            def accum(self, j):
                self.pf.wait()
                self.comm[j % 4] = (
                    self.comm[j % 4].astype(_F32) + self.st[...]
                ).astype(_BF)

            def store(self, j):
                chunk = self.ag_chunks[(j - 6) // 2]
                self.st[...] = self.comm[j % 4].astype(_F32)
                cp = pltpu.make_async_copy(
                    self.st,
                    out_ref.at[subrows(chunk, j % 2), self.col],
                    self.dma_sem)
                cp.start()
                cp.wait()

            def sig(self):
                pl.semaphore_signal(self.credit_sem, inc=1,
                                    device_id=(self.upstream,),
                                    device_id_type=_MESH)

            def take(self):
                pl.semaphore_wait(self.credit_sem, 1)

        R = Ring(commR, sendR, recvR, stR, cvR, dmaR, creditR,
                 target=right, upstream=left, col=half0,
                 init_chunk=ck(0), rs_chunk=lambda h: ck(-h - 1),
                 ag_chunks=[ck(0), ck(-1), ck(2)])
        L = Ring(commL, sendL, recvL, stL, cvL, dmaL, creditL,
                 target=left, upstream=right, col=half1,
                 init_chunk=ck(2), rs_chunk=lambda h: ck(h + 3),
                 ag_chunks=[ck(2), ck(3), ck(0)])
        rings = (R, L)

        for r in rings:
            r.prefetch(partial_ref.at[0, subrows(r.init_chunk, 0), r.col])
        for r in rings:
            r.pf.wait()
            r.cv[0] = r.st[...].astype(_BF)
            r.start(0)
            r.prefetch(partial_ref.at[0, subrows(r.init_chunk, 1), r.col])
        for r in rings:
            r.pf.wait()
            r.cv[1] = r.st[...].astype(_BF)
            r.start(1)
            r.pf_accum(0)

        for r in rings:
            r.wait_recv(0)
            r.accum(0)
            r.start(2)
            r.pf_accum(1)
        for r in rings:
            r.wait_recv(1)
            r.accum(1)
            r.start(3)
            r.pf_accum(2)
        for r in rings:
            r.wait_recv(2)
            r.accum(2)
            r.wait_send(2)
            r.sig()
            r.pf_accum(3)
        own = ck(1)
        for r in rings:
            r.wait_recv(3)
            r.accum(3)
            r.wait_send(3)
            r.sig()
            r.prefetch(partial_ref.at[0, subrows(own, 0), r.col])
        for r in rings:
            r.wait_send(0)
            r.take()
            r.start(4)
        for r in rings:
            r.wait_send(1)
            r.take()
            r.start(5)

        def epilogue(s):
            rsub = pl.ds(s * C2, C2)
            R.pf.wait()
            acc[rsub, half0] = commR[s].astype(_F32) + stR[...]
            R.prefetch(resid_ref.at[subrows(own, s), half0])
            L.pf.wait()
            acc[rsub, half1] = commL[s].astype(_F32) + stL[...]
            L.prefetch(resid_ref.at[subrows(own, s), half1])
            R.pf.wait()
            acc[rsub, half0] = acc[rsub, half0] + stR[...]
            L.pf.wait()
            acc[rsub, half1] = acc[rsub, half1] + stL[...]
            for t in range(C2 // TC):
                tr = pl.ds(s * C2 + t * TC, TC)
                y = acc[tr, :]
                ms = jnp.mean(y * y, axis=1, keepdims=True)
                acc[tr, :] = y * lax.rsqrt(ms + 1e-6) * gamma_ref[...]

        for r in rings:
            r.wait_recv(4)
        epilogue(0)
        for r in rings:
            r.wait_send(4)
            r.sig()
        for r in rings:
            r.cv[0] = acc[pl.ds(0, C2), r.col].astype(_BF)
            r.take()
            r.start(6)
            r.prefetch(partial_ref.at[0, subrows(own, 1), r.col])
        own0 = pltpu.make_async_copy(acc.at[pl.ds(0, C2), :],
                                     out_ref.at[subrows(own, 0), :], outR)
        own0.start()
        for r in rings:
            r.wait_recv(5)
        epilogue(1)
        for r in rings:
            r.wait_send(5)
            r.sig()
            r.sig()
        for r in rings:
            r.cv[1] = acc[pl.ds(C2, C2), r.col].astype(_BF)
            r.take()
            r.start(7)
        own1 = pltpu.make_async_copy(acc.at[pl.ds(C2, C2), :],
                                     out_ref.at[subrows(own, 1), :], outL)
        own1.start()
        for r in rings:
            r.sig()

        for r in rings:
            r.wait_recv(6)
            r.take()
            r.start(8)
            r.store(6)
        for r in rings:
            r.wait_recv(7)
            r.take()
            r.start(9)
            r.store(7)
        for r in rings:
            r.wait_send(8)
            r.sig()
        for r in rings:
            r.wait_recv(8)
            r.wait_send(6)
            r.take()
            r.start(10)
            r.store(8)
        for r in rings:
            r.wait_send(9)
            r.sig()
        for r in rings:
            r.wait_recv(9)
            r.wait_send(7)
            r.take()
            r.start(11)
            r.store(9)
        for r in rings:
            r.wait_recv(10)
            r.store(10)
        for r in rings:
            r.wait_recv(11)
            r.store(11)
        for r in rings:
            r.wait_send(10)
            r.wait_send(11)
        own0.wait()
        own1.wait()

    return pl.pallas_call(
        body,
        out_shape=jax.ShapeDtypeStruct((M, D), jnp.float32),
        in_specs=[
            pl.BlockSpec(memory_space=_ANY),
            pl.BlockSpec(memory_space=_ANY),
            pl.BlockSpec(memory_space=pltpu.VMEM),
        ],
        out_specs=pl.BlockSpec(memory_space=_ANY),
        scratch_shapes=[
            pltpu.VMEM((C, D), _F32),
            pltpu.VMEM((4, C2, H), _BF),
            pltpu.VMEM((4, C2, H), _BF),
            pltpu.VMEM((C2, H), _F32),
            pltpu.VMEM((C2, H), _F32),
            pltpu.VMEM((2, C2, H), _BF),
            pltpu.VMEM((2, C2, H), _BF),
            pltpu.SemaphoreType.DMA((4,)),
            pltpu.SemaphoreType.DMA((4,)),
            pltpu.SemaphoreType.DMA((4,)),
            pltpu.SemaphoreType.DMA((4,)),
            pltpu.SemaphoreType.DMA,
            pltpu.SemaphoreType.DMA,
            pltpu.SemaphoreType.DMA,
            pltpu.SemaphoreType.DMA,
            pltpu.SemaphoreType.REGULAR,
            pltpu.SemaphoreType.REGULAR,
        ],
        compiler_params=pltpu.CompilerParams(
            collective_id=0,
            vmem_limit_bytes=62 * 1024 * 1024,
        ),
    )(partial, resid, gamma2)
